# Initial kernel scaffold; baseline (speedup 1.0000x reference)
import jax
import jax.numpy as jnp
from jax.experimental import pallas as pl


def kernel(fnode_g, fmess_g_src, fmess_g_bond, agraph_g, bgraph_g, fnode_t, fmess_t_src, agraph_t, bgraph_t, cgraph, dgraph, revise_nodes, emb_table, W_a_w, W_a_b, outA_w, outA_b, W_i_w, W_i_b, W_g_w, W_g_b, W_t_w, W_t_b, outN_w, outN_b):
    raise NotImplementedError("write your pallas kernel here")


# speedup vs baseline: 2.4028x; 2.4028x over previous
import functools

import jax
import jax.numpy as jnp
from jax import lax
from jax.experimental import pallas as pl
from jax.experimental.pallas import tpu as pltpu
from jax.experimental.pallas import tpu_sc as plsc

NC = 2
NS = 16
NW = NC * NS
H = 128


def _gsum_call(tables, idxT, G):
    M = len(tables)
    K, Ep = idxT.shape
    Ew = Ep // NW
    nblk = Ew // G
    mesh = plsc.VectorSubcoreMesh(core_axis_name="c", subcore_axis_name="s")

    @functools.partial(
        pl.kernel,
        out_type=[jax.ShapeDtypeStruct((Ep, H), jnp.float32) for _ in range(M)],
        mesh=mesh,
        scratch_types=[
            pltpu.VMEM((K, G), jnp.int32),
            pltpu.VMEM((K, G, H), jnp.float32),
            pltpu.SemaphoreType.DMA,
        ],
    )
    def k(*refs):
        table_refs = refs[:M]
        idxT_hbm = refs[M]
        out_refs = refs[M + 1:M + 1 + M]
        idx_v, buf_v, sem = refs[M + 1 + M:]
        wid = lax.axis_index("s") * NC + lax.axis_index("c")

        def body(i, carry):
            base = wid * Ew + i * G
            for j in range(K):
                pltpu.sync_copy(idxT_hbm.at[j, pl.ds(base, G)], idx_v.at[j])
            for m in range(M):
                cps = [
                    pltpu.async_copy(table_refs[m].at[idx_v.at[j]], buf_v.at[j], sem)
                    for j in range(K)
                ]
                for cp in cps:
                    cp.wait()
                if K > 1:
                    def row(r, c2):
                        for cg in range(H // 16):
                            s = pl.ds(cg * 16, 16)
                            acc = buf_v[0, r, s]
                            for j in range(1, K):
                                acc = acc + buf_v[j, r, s]
                            buf_v[0, r, s] = acc
                        return c2
                    lax.fori_loop(0, G, row, 0, unroll=False)
                pltpu.sync_copy(buf_v.at[0], out_refs[m].at[pl.ds(base, G)])
            return carry

        lax.fori_loop(0, nblk, body, 0, unroll=False)

    return k(*tables, idxT)


def _gathersum(tables, idx2d, G):
    E, K = idx2d.shape
    chunk = NW * G
    Ep = ((E + chunk - 1) // chunk) * chunk
    idxT = jnp.transpose(idx2d)
    if Ep != E:
        idxT = jnp.pad(idxT, ((0, 0), (0, Ep - E)))
    outs = _gsum_call(tables, idxT, G)
    return [o[:E] for o in outs]


def _fused_mm(E, xs, ws, codes, T, adds, b, relu, zero0, pair=False, BR=1000):
    NB = E // BR
    n = len(xs)
    na = len(adds)
    have_oh = codes is not None
    args = []
    in_specs = []
    if have_oh:
        V = T.shape[0]
        args.append(codes.reshape(NB, 1, BR))
        in_specs.append(pl.BlockSpec((1, 1, BR), lambda i: (i, 0, 0)))
        args.append(T)
        in_specs.append(pl.BlockSpec(T.shape, lambda i: (0, 0)))
    for x, w in zip(xs, ws):
        args.append(x)
        in_specs.append(pl.BlockSpec((BR, H), lambda i: (i, 0)))
        args.append(w)
        in_specs.append(pl.BlockSpec((H, H), lambda i: (0, 0)))
    for a in adds:
        args.append(a)
        in_specs.append(pl.BlockSpec((BR, H), lambda i: (i, 0)))
    if b is not None:
        args.append(b.reshape(1, H))
        in_specs.append(pl.BlockSpec((1, H), lambda i: (0, 0)))

    def body(*refs):
        pos = 0
        acc = None
        if have_oh:
            c = refs[0][0, 0, :]
            t = refs[1][...]
            oh = (c[:, None] == lax.broadcasted_iota(jnp.int32, (BR, V), 1))
            acc = jnp.dot(oh.astype(jnp.float32), t,
                          preferred_element_type=jnp.float32)
            pos = 2
        for i in range(n):
            p = jnp.dot(refs[pos + 2 * i][...], refs[pos + 2 * i + 1][...],
                        preferred_element_type=jnp.float32)
            acc = p if acc is None else acc + p
        pos += 2 * n
        for i in range(na):
            acc = refs[pos + i][...] if acc is None else acc + refs[pos + i][...]
        pos += na
        if b is not None:
            acc = acc + refs[pos][...]
            pos += 1
        raw = acc
        proc = jnp.maximum(raw, 0.0) if relu else raw
        if zero0:
            pid = pl.program_id(0)
            rows = lax.broadcasted_iota(jnp.int32, (BR, H), 0)
            proc = jnp.where((pid == 0) & (rows == 0), 0.0, proc)
        if pair:
            refs[pos][...] = raw
            refs[pos + 1][...] = proc
        else:
            refs[pos][...] = proc

    out_spec = pl.BlockSpec((BR, H), lambda i: (i, 0))
    shp = jax.ShapeDtypeStruct((E, H), jnp.float32)
    if pair:
        out_shape = [shp, shp]
        out_specs = [out_spec, out_spec]
    else:
        out_shape = shp
        out_specs = out_spec
    return pl.pallas_call(
        body,
        grid=(NB,),
        in_specs=in_specs,
        out_specs=out_specs,
        out_shape=out_shape,
    )(*args)


def kernel(fnode_g, fmess_g_src, fmess_g_bond, agraph_g, bgraph_g, fnode_t,
           fmess_t_src, agraph_t, bgraph_t, cgraph, dgraph, revise_nodes,
           emb_table, W_a_w, W_a_b, outA_w, outA_b, W_i_w, W_i_b, W_g_w,
           W_g_b, W_t_w, W_t_b, outN_w, outN_b):
    ATOM_SIZE = 40
    BOND_SIZE = 4
    DEPTH = 3

    Tcomb = (W_a_w[:ATOM_SIZE, None, :]
             + W_a_w[None, ATOM_SIZE:ATOM_SIZE + BOND_SIZE, :]
             ).reshape(ATOM_SIZE * BOND_SIZE, H)
    Wa_bot = W_a_w[ATOM_SIZE + BOND_SIZE:]
    outA_top = outA_w[:ATOM_SIZE]
    Od = [outA_w[ATOM_SIZE + d * H:ATOM_SIZE + (d + 1) * H] for d in range(DEPTH)]
    W_i_top, W_i_bot = W_i_w[:H], W_i_w[H:]
    W_t_top, W_t_bot = W_t_w[:H], W_t_w[H:]
    Wc1 = W_g_w[:H] @ W_t_top
    Wc2 = W_g_w[H:] @ W_t_top
    bc = W_g_b @ W_t_top + W_t_b
    Nd = [outN_w[d * H:(d + 1) * H] for d in range(DEPTH + 1)]

    c_g = jnp.take(fnode_g, fmess_g_src) * BOND_SIZE + fmess_g_bond

    N_GMESS = bgraph_g.shape[0]
    N_ATOM = agraph_g.shape[0]
    N_NODE = agraph_t.shape[0]
    N_TMESS = bgraph_t.shape[0]

    m1 = _fused_mm(N_GMESS, [], [], c_g, Tcomb, [], W_a_b, True, True)
    nei = _gathersum([m1], bgraph_g, 64)[0]
    m2 = _fused_mm(N_GMESS, [nei], [Wa_bot], c_g, Tcomb, [], W_a_b, True, True)
    nei = _gathersum([m2], bgraph_g, 64)[0]
    m3 = _fused_mm(N_GMESS, [nei], [Wa_bot], c_g, Tcomb, [], W_a_b, True, True)
    a1, a2, a3 = _gathersum([m1, m2, m3], agraph_g, 64)
    hatom = _fused_mm(N_ATOM, [a1, a2, a3], Od, fnode_g, outA_top, [],
                      outA_b, True, True)

    finput = _gathersum([emb_table], fnode_t[:, None], 64)[0]
    hnode_agg = _gathersum([hatom], dgraph, 32)[0]
    hmess2 = _gathersum([hatom], cgraph, 64)[0]
    hnode_t = _fused_mm(N_NODE, [finput, hnode_agg], [W_i_top, W_i_bot],
                        None, None, [], W_i_b, False, False)
    hmess1 = _gathersum([hnode_t], fmess_t_src[:, None], 64)[0]
    pre_t, mt1 = _fused_mm(N_TMESS, [hmess1, hmess2], [Wc1, Wc2], None, None,
                           [], bc, True, True, pair=True)

    neit = _gathersum([mt1], bgraph_t, 64)[0]
    mt2 = _fused_mm(N_TMESS, [neit], [W_t_bot], None, None, [pre_t], None,
                    True, True)
    neit = _gathersum([mt2], bgraph_t, 64)[0]
    mt3 = _fused_mm(N_TMESS, [neit], [W_t_bot], None, None, [pre_t], None,
                    True, True)
    t1, t2, t3 = _gathersum([mt1, mt2, mt3], agraph_t, 64)
    hnode = _fused_mm(N_NODE, [hnode_t, t1, t2, t3], Nd, None, None, [],
                      outN_b, True, True)

    embedding = _gathersum([hnode], revise_nodes, 32)[0]
    return embedding, hnode, hatom
